# Initial kernel scaffold; baseline (speedup 1.0000x reference)
import jax
import jax.numpy as jnp
from jax.experimental import pallas as pl


def kernel(token_ids, weight):
    raise NotImplementedError("write your pallas kernel here")


# speedup vs baseline: 1.8452x; 1.8452x over previous
import functools

import jax
import jax.numpy as jnp
from jax import lax
from jax.experimental import pallas as pl
from jax.experimental.pallas import tpu as pltpu
from jax.experimental.pallas import tpu_sc as plsc

IDX_W = 128
K = 8


def _flat_gather(idx2d, weight):
    nrows = idx2d.shape[0]
    d = weight.shape[1]
    info = plsc.get_sparse_core_info()
    nw = info.num_cores * info.num_subcores
    per_w = nrows // nw
    steps = per_w // K

    mesh = plsc.VectorSubcoreMesh(core_axis_name="c", subcore_axis_name="s")

    @functools.partial(
        pl.kernel,
        mesh=mesh,
        out_type=jax.ShapeDtypeStruct((nrows, IDX_W, d), jnp.float32),
        scratch_types=[
            pltpu.VMEM((K, IDX_W), jnp.int32),
            pltpu.VMEM((K, IDX_W, d), jnp.float32),
            pltpu.SemaphoreType.DMA,
        ],
        compiler_params=pltpu.CompilerParams(use_tc_tiling_on_sc=False),
    )
    def body(idx_hbm, table_hbm, out_hbm, idx_v, rows_v, sem):
        wid = lax.axis_index("s") * info.num_cores + lax.axis_index("c")
        base = wid * per_w

        def step(g, carry):
            row0 = base + g * K
            pltpu.sync_copy(idx_hbm.at[pl.ds(row0, K)], idx_v)
            copies = [
                pltpu.async_copy(table_hbm.at[idx_v.at[j]], rows_v.at[j], sem)
                for j in range(K)
            ]
            for cp in copies:
                cp.wait()
            pltpu.sync_copy(rows_v, out_hbm.at[pl.ds(row0, K)])
            return carry

        lax.fori_loop(0, steps, step, 0)

    return body(idx2d, weight)


def kernel(token_ids, weight):
    b, s = token_ids.shape
    d = weight.shape[1]
    n = b * s
    idx2d = token_ids.reshape(n // IDX_W, IDX_W)
    out = _flat_gather(idx2d, weight)
    return out.reshape(b, s, d)


# speedup vs baseline: 1.8727x; 1.0149x over previous
import functools

import jax
import jax.numpy as jnp
from jax import lax
from jax.experimental import pallas as pl
from jax.experimental.pallas import tpu as pltpu
from jax.experimental.pallas import tpu_sc as plsc

IDX_W = 128
NBUF = 8


def _flat_gather(idx2d, weight):
    nrows = idx2d.shape[0]
    d = weight.shape[1]
    info = plsc.get_sparse_core_info()
    nw = info.num_cores * info.num_subcores
    per_w = nrows // nw
    ngroups = per_w // NBUF

    mesh = plsc.VectorSubcoreMesh(core_axis_name="c", subcore_axis_name="s")

    @functools.partial(
        pl.kernel,
        mesh=mesh,
        out_type=jax.ShapeDtypeStruct((nrows, IDX_W, d), jnp.float32),
        scratch_types=[
            pltpu.VMEM((per_w, IDX_W), jnp.int32),
            pltpu.VMEM((NBUF, IDX_W, d), jnp.float32),
            pltpu.SemaphoreType.DMA((NBUF,)),
            pltpu.SemaphoreType.DMA((NBUF,)),
        ],
        compiler_params=pltpu.CompilerParams(use_tc_tiling_on_sc=False),
    )
    def body(idx_hbm, table_hbm, out_hbm, idx_v, rows_v, gsem, ssem):
        wid = lax.axis_index("s") * info.num_cores + lax.axis_index("c")
        base = wid * per_w

        pltpu.sync_copy(idx_hbm.at[pl.ds(base, per_w)], idx_v)

        for b in range(NBUF):
            pltpu.async_copy(table_hbm.at[idx_v.at[b]], rows_v.at[b], gsem.at[b])

        def group(g, carry):
            r0 = g * NBUF
            for b in range(NBUF):
                pltpu.make_async_copy(
                    table_hbm.at[idx_v.at[b]], rows_v.at[b], gsem.at[b]
                ).wait()
                pltpu.async_copy(rows_v.at[b], out_hbm.at[base + r0 + b], ssem.at[b])

            @pl.when(g + 1 < ngroups)
            def _refill():
                for b in range(NBUF):
                    pltpu.make_async_copy(
                        rows_v.at[b], out_hbm.at[base], ssem.at[b]
                    ).wait()
                    pltpu.async_copy(
                        table_hbm.at[idx_v.at[r0 + NBUF + b]],
                        rows_v.at[b],
                        gsem.at[b],
                    )

            return carry

        lax.fori_loop(0, ngroups, group, 0)

        for b in range(NBUF):
            pltpu.make_async_copy(rows_v.at[b], out_hbm.at[base], ssem.at[b]).wait()

    return body(idx2d, weight)


def kernel(token_ids, weight):
    b, s = token_ids.shape
    d = weight.shape[1]
    n = b * s
    idx2d = token_ids.reshape(n // IDX_W, IDX_W)
    out = _flat_gather(idx2d, weight)
    return out.reshape(b, s, d)
